# baseline (speedup 1.0000x reference)
import functools

import jax
import jax.numpy as jnp
from jax import lax
from jax.experimental import pallas as pl
from jax.experimental.pallas import tpu as pltpu
from jax.experimental.pallas import tpu_sc as plsc

B = 1024
L = 200
H = 128
POS_ROWS = 513
SEG_ROWS = 3
FUSED_ROWS = SEG_ROWS * POS_ROWS
N = B * L
NW = 32
PER_W = N // NW
C = 32
NCHUNK = PER_W // C
NGRP = C // 16
NCOL = H // 16
MPAIR = 33


def _fused_tc_body(tok_ref, pos_ref, seg_ref, out_ref):
    tp = tok_ref[...] + pos_ref[...]
    out_ref[...] = tp[None, :, :] + seg_ref[...][:, None, :]


def _sc_body(x_hbm, xseg_hbm, fused_hbm, tokpos_hbm, seg_hbm, out_hbm,
             xi, si, tp, sg, civ2d,
             v0, v1, g1, g2, g3, g4,
             sv0, sv1, sg1, sg2, sg3, sg4):
    wid = lax.axis_index("s") * 2 + lax.axis_index("c")
    base = wid * PER_W
    pltpu.sync_copy(x_hbm.at[wid], xi)
    pltpu.sync_copy(xseg_hbm.at[wid], si)
    pltpu.sync_copy(tokpos_hbm, tp)
    pltpu.sync_copy(seg_hbm, sg)

    segv = [[sg[s, pl.ds(j * 16, 16)] for j in range(NCOL)]
            for s in range(SEG_ROWS)]
    gbuf = {1: g1, 2: g2, 3: g3, 4: g4}
    gsem = {1: sg1, 2: sg2, 3: sg3, 4: sg4}
    vbuf = (v0, v1)
    vsem = (sv0, sv1)

    def idx_slice(arr, i, g):
        q = i * NGRP + g
        return arr[q >> 3, pl.ds((q & 7) * 16, 16)]

    def issue(i, slot):
        for j in range(NGRP):
            civ2d[slot - 1, pl.ds(j * 16, 16)] = (
                idx_slice(si, i, j) * POS_ROWS + idx_slice(xi, i, j))
        pltpu.async_copy(
            fused_hbm.at[civ2d.at[slot - 1]], gbuf[slot], gsem[slot])

    def wait_gather(slot):
        pltpu.make_async_copy(
            fused_hbm.at[civ2d.at[slot - 1]], gbuf[slot], gsem[slot]).wait()

    def gout(i, slot):
        pltpu.async_copy(
            gbuf[slot], out_hbm.at[pl.ds(base + i * C, C)], gsem[slot])

    def wait_gout(slot):
        pltpu.make_async_copy(
            gbuf[slot], out_hbm.at[pl.ds(base, C)], gsem[slot]).wait()

    def vout(i, vs):
        pltpu.async_copy(
            vbuf[vs], out_hbm.at[pl.ds(base + i * C, C)], vsem[vs])

    def wait_vout(vs):
        pltpu.make_async_copy(
            vbuf[vs], out_hbm.at[pl.ds(base, C)], vsem[vs]).wait()

    def expand(i, vs):
        st = vbuf[vs]

        @plsc.parallel_loop(0, NGRP, unroll=2)
        def grp(g):
            pvec = idx_slice(xi, i, g)
            svec = idx_slice(si, i, g)
            for l in range(16):
                p = pvec[l]
                s = svec[l]
                m1 = s == 1
                m2 = s == 2
                for j in range(NCOL):
                    tv = tp[p, pl.ds(j * 16, 16)]
                    sv = jnp.where(m2, segv[2][j],
                                   jnp.where(m1, segv[1][j], segv[0][j]))
                    st[g * 16 + l, pl.ds(j * 16, 16)] = tv + sv

    issue(1, 1)
    issue(2, 2)

    def step(m, carry):
        i0 = 6 * m

        @pl.when(m > 0)
        def _():
            wait_vout(0)

        expand(i0, 0)
        vout(i0, 0)

        @pl.when(m > 0)
        def _():
            wait_gout(3)
            wait_gout(4)

        issue(i0 + 4, 3)
        issue(i0 + 5, 4)
        wait_gather(1)
        gout(i0 + 1, 1)
        wait_gather(2)
        gout(i0 + 2, 2)

        @pl.when(m > 0)
        def _():
            wait_vout(1)

        expand(i0 + 3, 1)
        vout(i0 + 3, 1)
        wait_gout(1)
        issue(i0 + 7, 1)
        wait_gout(2)

        @pl.when(i0 + 8 < NCHUNK)
        def _():
            issue(i0 + 8, 2)

        wait_gather(3)
        gout(i0 + 4, 3)
        wait_gather(4)
        gout(i0 + 5, 4)
        return carry

    lax.fori_loop(0, MPAIR, step, 0)

    wait_vout(0)
    expand(NCHUNK - 2, 0)
    vout(NCHUNK - 2, 0)
    wait_gather(1)
    gout(NCHUNK - 1, 1)
    wait_vout(0)
    wait_vout(1)
    wait_gout(1)
    wait_gout(3)
    wait_gout(4)


@jax.jit
def _run(x3d, xseg3d, tok513, pos_table, seg_table):
    fused = pl.pallas_call(
        _fused_tc_body,
        out_shape=jax.ShapeDtypeStruct((SEG_ROWS, POS_ROWS, H), jnp.float32),
    )(tok513, pos_table, seg_table)
    ff = fused.reshape(FUSED_ROWS, H)
    tokpos = ff[:POS_ROWS]

    mesh = plsc.VectorSubcoreMesh(core_axis_name="c", subcore_axis_name="s")
    call = pl.kernel(
        _sc_body,
        out_type=jax.ShapeDtypeStruct((N, H), jnp.float32),
        mesh=mesh,
        scratch_types=[
            pltpu.VMEM((NCHUNK * C // 128, 128), jnp.int32),
            pltpu.VMEM((NCHUNK * C // 128, 128), jnp.int32),
            pltpu.VMEM((POS_ROWS, H), jnp.float32),
            pltpu.VMEM((SEG_ROWS, H), jnp.float32),
            pltpu.VMEM((4, C), jnp.int32),
            pltpu.VMEM((C, H), jnp.float32),
            pltpu.VMEM((C, H), jnp.float32),
            pltpu.VMEM((C, H), jnp.float32),
            pltpu.VMEM((C, H), jnp.float32),
            pltpu.VMEM((C, H), jnp.float32),
            pltpu.VMEM((C, H), jnp.float32),
            pltpu.SemaphoreType.DMA,
            pltpu.SemaphoreType.DMA,
            pltpu.SemaphoreType.DMA,
            pltpu.SemaphoreType.DMA,
            pltpu.SemaphoreType.DMA,
            pltpu.SemaphoreType.DMA,
        ],
    )
    return call(x3d, xseg3d, ff, tokpos, seg_table)


def kernel(x, x_seg, token_table, pos_table, seg_table):
    x3d = x.reshape(NW, NCHUNK * C // 128, 128)
    xseg3d = x_seg.reshape(NW, NCHUNK * C // 128, 128)
    out = _run(x3d, xseg3d, token_table[:POS_ROWS], pos_table, seg_table)
    return out.reshape(B, L, H)


# speedup vs baseline: 1.4708x; 1.4708x over previous
import functools

import jax
import jax.numpy as jnp
from jax import lax
from jax.experimental import pallas as pl
from jax.experimental.pallas import tpu as pltpu
from jax.experimental.pallas import tpu_sc as plsc

B = 1024
L = 200
H = 128
POS_ROWS = 513
SEG_ROWS = 3
N = B * L
NW = 32
PER_W = N // NW
C = 128
NCHUNK = PER_W // C
NBUF = 6
NCOL = H // 16


def _fused_tc_body(tok_ref, pos_ref, seg_ref, out_ref):
    tp = tok_ref[...] + pos_ref[...]
    out_ref[...] = tp[None, :, :] + seg_ref[...][:, None, :]


def _sc_body(x_hbm, xseg_hbm, fused_hbm, out_hbm,
             xi, si, b0, b1, b2, b3, b4, b5,
             sg0, sg1, sg2, sg3, sg4, sg5,
             so0, so1, so2, so3, so4, so5):
    wid = lax.axis_index("s") * 2 + lax.axis_index("c")
    base = wid * PER_W
    pltpu.sync_copy(x_hbm.at[wid], xi)
    pltpu.sync_copy(xseg_hbm.at[wid], si)

    def mkidx(r, carry):
        for j in range(NCOL):
            sl = (r, pl.ds(j * 16, 16))
            si[sl] = si[sl] * POS_ROWS + xi[sl]
        return carry

    lax.fori_loop(0, NCHUNK, mkidx, 0)

    bufs = (b0, b1, b2, b3, b4, b5)
    sgs = (sg0, sg1, sg2, sg3, sg4, sg5)
    sos = (so0, so1, so2, so3, so4, so5)

    def issue(i, b):
        pltpu.async_copy(fused_hbm.at[si.at[i]], bufs[b], sgs[b])

    def wait_gather(b):
        pltpu.make_async_copy(fused_hbm.at[si.at[0]], bufs[b], sgs[b]).wait()

    def wait_out(b):
        pltpu.make_async_copy(
            bufs[b], out_hbm.at[pl.ds(base, C)], sos[b]).wait()

    for b in range(NBUF - 1):
        issue(b, b)

    def step(k, carry):
        for b in range(NBUF):
            i = NBUF * k + b
            wait_gather(b)
            pltpu.async_copy(
                bufs[b], out_hbm.at[pl.ds(base + i * C, C)], sos[b])
            nxt = (b + NBUF - 1) % NBUF

            @pl.when(NBUF * k + b + NBUF - 1 < NCHUNK)
            def _():
                @pl.when(k + b > 0)
                def _():
                    wait_out(nxt)

                issue(i + NBUF - 1, nxt)
        return carry

    lax.fori_loop(0, NCHUNK // NBUF, step, 0)
    for i in range(NBUF * (NCHUNK // NBUF), NCHUNK):
        b = i % NBUF
        wait_gather(b)
        pltpu.async_copy(
            bufs[b], out_hbm.at[pl.ds(base + i * C, C)], sos[b])
    for i in range(NCHUNK - NBUF, NCHUNK):
        wait_out(i % NBUF)


@jax.jit
def _run(x3d, xseg3d, tok513, pos_table, seg_table):
    fused = pl.pallas_call(
        _fused_tc_body,
        out_shape=jax.ShapeDtypeStruct((SEG_ROWS, POS_ROWS, H), jnp.float32),
    )(tok513, pos_table, seg_table)
    fused = fused.reshape(SEG_ROWS * POS_ROWS, H)

    mesh = plsc.VectorSubcoreMesh(core_axis_name="c", subcore_axis_name="s")
    call = pl.kernel(
        _sc_body,
        out_type=jax.ShapeDtypeStruct((N, H), jnp.float32),
        mesh=mesh,
        scratch_types=[
            pltpu.VMEM((NCHUNK, C), jnp.int32),
            pltpu.VMEM((NCHUNK, C), jnp.int32),
            pltpu.VMEM((C, H), jnp.float32),
            pltpu.VMEM((C, H), jnp.float32),
            pltpu.VMEM((C, H), jnp.float32),
            pltpu.VMEM((C, H), jnp.float32),
            pltpu.VMEM((C, H), jnp.float32),
            pltpu.VMEM((C, H), jnp.float32),
            pltpu.SemaphoreType.DMA,
            pltpu.SemaphoreType.DMA,
            pltpu.SemaphoreType.DMA,
            pltpu.SemaphoreType.DMA,
            pltpu.SemaphoreType.DMA,
            pltpu.SemaphoreType.DMA,
            pltpu.SemaphoreType.DMA,
            pltpu.SemaphoreType.DMA,
            pltpu.SemaphoreType.DMA,
            pltpu.SemaphoreType.DMA,
            pltpu.SemaphoreType.DMA,
            pltpu.SemaphoreType.DMA,
        ],
    )
    return call(x3d, xseg3d, fused)


def kernel(x, x_seg, token_table, pos_table, seg_table):
    x3d = x.reshape(NW, NCHUNK, C)
    xseg3d = x_seg.reshape(NW, NCHUNK, C)
    out = _run(x3d, xseg3d, token_table[:POS_ROWS], pos_table, seg_table)
    return out.reshape(B, L, H)
